# baseline (speedup 1.0000x reference)
import functools

import jax
import jax.numpy as jnp
from jax import lax
from jax.experimental import pallas as pl
from jax.experimental.pallas import tpu as pltpu
from jax.experimental.pallas import tpu_sc as plsc

N = 10000
E = 160000
IN_DIM = 128
EMB = 256
HEADS = 4
LAYERS = 3
EDGE_DIM = 16
B = 64

FW = 256
NW = 32
NPT = 96
SWEEPS = 4
N2 = NPT * NW * SWEEPS
NSEG = NW * SWEEPS
E3 = E + 1024
BLK = 512
ACCW = NPT * HEADS * FW
DENW = NPT * HEADS * 16


def _linear_relu_kernel(x_ref, w_ref, b_ref, o_ref):
    o_ref[...] = jax.nn.relu(
        jnp.dot(x_ref[...], w_ref[...], preferred_element_type=jnp.float32)
        + b_ref[...]
    )


def _linear_relu(x, w, b):
    n, k = x.shape
    m = w.shape[1]
    blk = 1000
    return pl.pallas_call(
        _linear_relu_kernel,
        out_shape=jax.ShapeDtypeStruct((n, m), jnp.float32),
        grid=(n // blk,),
        in_specs=[
            pl.BlockSpec((blk, k), lambda i: (i, 0)),
            pl.BlockSpec((k, m), lambda i: (0, 0)),
            pl.BlockSpec((m,), lambda i: (0,)),
        ],
        out_specs=pl.BlockSpec((blk, m), lambda i: (i, 0)),
    )(x, w, b)


_GDN = lax.GatherDimensionNumbers(
    offset_dims=(), collapsed_slice_dims=(0,), start_index_map=(0,))


def _dyn_gather(v, idx):
    return lax.gather(v, idx[:, None], _GDN, (1,),
                      mode=lax.GatherScatterMode.PROMISE_IN_BOUNDS)


def _splat(v, j):
    return _dyn_gather(v, jnp.full((16,), j, jnp.int32))


def _lane(v, j):
    return jnp.squeeze(lax.slice(v, (j,), (j + 1,)))


def _make_sc_agg():
    mesh = plsc.VectorSubcoreMesh(core_axis_name="c", subcore_axis_name="s")

    @functools.partial(
        pl.kernel,
        mesh=mesh,
        out_type=[jax.ShapeDtypeStruct((N2 * HEADS * FW,), jnp.float32),
                  jax.ShapeDtypeStruct((N2 * HEADS * 16,), jnp.float32)],
        scratch_types=[
            pltpu.VMEM((BLK,), jnp.int32),
            pltpu.VMEM((BLK,), jnp.int32),
            pltpu.VMEM((BLK * HEADS,), jnp.float32),
            pltpu.VMEM((16,), jnp.int32),
            pltpu.VMEM((16, FW), jnp.float32),
            pltpu.VMEM((ACCW,), jnp.float32),
            pltpu.VMEM((DENW,), jnp.float32),
            pltpu.SemaphoreType.DMA,
        ],
    )
    def sc_agg(src_h, dst_h, w_h, bounds_h, x_h, out_h, dout_h,
               sblk_v, dblk_v, wblk_v, bnd_v, hrow_v, acc_v, den_v, sem):
        cc = lax.axis_index("c")
        ss = lax.axis_index("s")
        t = cc * 16 + ss
        lane = lax.iota(jnp.int32, 16)
        z16f = jnp.zeros((16,), jnp.float32)

        def zacc_body(i, carry):
            acc_v[pl.ds(i * 16, 16)] = z16f
            return carry

        def zden_body(i, carry):
            den_v[pl.ds(i * 16, 16)] = z16f
            return carry
        lax.fori_loop(0, ACCW // 16, zacc_body, 0)
        lax.fori_loop(0, DENW // 16, zden_body, 0)

        pltpu.sync_copy(bounds_h.at[t], bnd_v)
        bv = bnd_v[pl.ds(0, 16)]
        sts = [_lane(bv, 2 * j) for j in range(SWEEPS)]
        ens = [_lane(bv, 2 * j + 1) for j in range(SWEEPS)]

        def sweep_body(sw, carry0):
            seg = sw * NW + t
            st = sts[0]
            en = ens[0]
            for j in range(1, SWEEPS):
                st = jnp.where(sw == j, sts[j], st)
                en = jnp.where(sw == j, ens[j], en)
            lo = seg * NPT
            hi = lo + NPT

            astart = pl.multiple_of((st // 8) * 8, 8)
            nblk = (en - astart + BLK - 1) // BLK

            def blk_body(k, carry1):
                eo = pl.multiple_of(astart + k * BLK, 8)
                pltpu.sync_copy(src_h.at[pl.ds(eo, BLK)], sblk_v)
                pltpu.sync_copy(dst_h.at[pl.ds(eo, BLK)], dblk_v)
                pltpu.sync_copy(
                    w_h.at[pl.ds(eo * HEADS, BLK * HEADS)], wblk_v)

                def chunk_body(i, carry2):
                    io = i * 16
                    s16 = sblk_v[pl.ds(io, 16)]
                    d16 = dblk_v[pl.ds(io, 16)]
                    ld = jnp.clip(d16, lo, hi - 1) - lo
                    pos = eo + io + lane
                    vf = jnp.where((pos >= st) & (pos < en), 1.0, 0.0)
                    cp = pltpu.async_copy(x_h.at[s16], hrow_v, sem)
                    wv = [wblk_v[pl.ds((io + p * 4) * HEADS, 16)]
                          for p in range(4)]
                    cp.wait()
                    for q in range(16):
                        ldq = jnp.squeeze(lax.slice(ld, (q,), (q + 1,)))
                        slot = ldq * (HEADS * FW)
                        dslot = ldq * (HEADS * 16)
                        nm = _splat(vf, q)
                        wq = wv[q // 4]
                        for h_ in range(HEADS):
                            wsp = _splat(wq, (q % 4) * HEADS + h_) * nm
                            ab = pl.multiple_of(slot + h_ * FW, 16)
                            db = pl.multiple_of(dslot + h_ * 16, 16)
                            den_v[pl.ds(db, 16)] = (
                                den_v[pl.ds(db, 16)]
                                + jnp.where(lane == 0, wsp, 0.0))
                            for f in range(FW // 16):
                                hvf = hrow_v[q, pl.ds(f * 16, 16)]
                                acc_v[pl.ds(ab + f * 16, 16)] = (
                                    acc_v[pl.ds(ab + f * 16, 16)]
                                    + wsp * hvf)
                    return carry2
                return lax.fori_loop(0, BLK // 16, chunk_body, carry1)
            lax.fori_loop(0, nblk, blk_body, 0)

            ob = pl.multiple_of(lo * (HEADS * FW), 8192)
            pltpu.sync_copy(acc_v, out_h.at[pl.ds(ob, ACCW)])
            dob = pl.multiple_of(lo * (HEADS * 16), 2048)
            pltpu.sync_copy(den_v, dout_h.at[pl.ds(dob, DENW)])
            lax.fori_loop(0, ACCW // 16, zacc_body, 0)
            lax.fori_loop(0, DENW // 16, zden_body, 0)
            return carry0
        lax.fori_loop(0, SWEEPS, sweep_body, 0)

    return sc_agg


_sc_agg = _make_sc_agg()


def _leaky(a):
    return jnp.where(a >= 0, a, 0.2 * a)


def kernel(x, edge_index, edge_attr, batch, params):
    src = edge_index[0]
    dst = edge_index[1]

    perm = jnp.argsort(dst).astype(jnp.int32)
    dsts = dst[perm]
    srcs = src[perm]
    nb = (jnp.arange(NSEG + 1, dtype=jnp.int32) * NPT).clip(0, N)
    starts = jnp.searchsorted(dsts, nb).astype(jnp.int32)
    s1 = starts[:NSEG].reshape(SWEEPS, NW).T
    s2 = starts[1:NSEG + 1].reshape(SWEEPS, NW).T
    bounds = jnp.concatenate(
        [jnp.stack([s1, s2], axis=2).reshape(NW, 2 * SWEEPS),
         jnp.zeros((NW, 8), jnp.int32)], axis=1)
    epad = E3 - E
    srcs_p = jnp.concatenate([srcs, jnp.zeros((epad,), jnp.int32)])
    dsts_p = jnp.concatenate([dsts, jnp.full((epad,), N - 1, jnp.int32)])

    h = _linear_relu(x, params['W0'], params['b0'])

    ea_mean = jnp.mean(edge_attr, axis=0)
    outs = []
    for lp in params['layers']:
        W = lp['W'].reshape(EMB, HEADS, EMB)
        ws = jnp.einsum('dhk,hk->dh', W, lp['att_src'])
        wd = jnp.einsum('dhk,hk->dh', W, lp['att_dst'])
        ve = jnp.einsum('dhk,hk->dh',
                        lp['We'].reshape(EDGE_DIM, HEADS, EMB), lp['att_e'])

        as_ = h @ ws
        ad_ = h @ wd
        aev = edge_attr @ ve
        ae_loop = ea_mean @ ve

        a = as_[srcs] + ad_[dsts] + aev[perm]
        w_e = jnp.exp(_leaky(a))
        w_l = jnp.exp(_leaky(as_ + ad_ + ae_loop))

        w_p = jnp.concatenate(
            [w_e, jnp.zeros((epad, HEADS), jnp.float32)]).reshape(-1)
        sc_acc, sc_den = _sc_agg(srcs_p, dsts_p, w_p, bounds, h)
        acc = sc_acc.reshape(N2, HEADS, EMB)[:N]
        den = sc_den.reshape(N2, HEADS, 16)[:N, :, 0]
        acc = acc + w_l[:, :, None] * h[:, None, :]
        den = den + w_l + 1e-16
        z = acc / den[:, :, None]

        out = jnp.einsum('nhd,dhk->nk', z, W) / HEADS + lp['bias']

        m = jnp.mean(out, 0)
        v = jnp.var(out, 0)
        out = (out - m) / jnp.sqrt(v + 1e-5) * lp['gamma'] + lp['beta']
        h = h + jax.nn.relu(out)
        outs.append(h)

    seg = jax.nn.one_hot(batch, B, dtype=jnp.float32)
    pooled = [seg.T @ o for o in outs]
    zs = jnp.concatenate(pooled, axis=1)
    gates = jax.nn.softmax(zs @ params['Wg'] + params['bg'], axis=1)
    zt = jnp.stack(pooled, axis=1)
    z = jnp.sum(zt * gates[..., None], axis=1)
    return (z, outs[-1])


# speedup vs baseline: 1.2092x; 1.2092x over previous
import functools

import jax
import jax.numpy as jnp
from jax import lax
from jax.experimental import pallas as pl
from jax.experimental.pallas import tpu as pltpu
from jax.experimental.pallas import tpu_sc as plsc

N = 10000
E = 160000
IN_DIM = 128
EMB = 256
HEADS = 4
LAYERS = 3
EDGE_DIM = 16
B = 64

FW = 256
NW = 32
NPT = 88
SWEEPS = 4
N2 = NPT * NW * SWEEPS
NSEG = NW * SWEEPS
E3 = E + 1024
BLK = 128
ACCW = NPT * HEADS * FW
DENW = NPT * HEADS * 16


def _linear_relu_kernel(x_ref, w_ref, b_ref, o_ref):
    o_ref[...] = jax.nn.relu(
        jnp.dot(x_ref[...], w_ref[...], preferred_element_type=jnp.float32)
        + b_ref[...]
    )


def _linear_relu(x, w, b):
    n, k = x.shape
    m = w.shape[1]
    blk = 1000
    return pl.pallas_call(
        _linear_relu_kernel,
        out_shape=jax.ShapeDtypeStruct((n, m), jnp.float32),
        grid=(n // blk,),
        in_specs=[
            pl.BlockSpec((blk, k), lambda i: (i, 0)),
            pl.BlockSpec((k, m), lambda i: (0, 0)),
            pl.BlockSpec((m,), lambda i: (0,)),
        ],
        out_specs=pl.BlockSpec((blk, m), lambda i: (i, 0)),
    )(x, w, b)


_GDN = lax.GatherDimensionNumbers(
    offset_dims=(), collapsed_slice_dims=(0,), start_index_map=(0,))


def _dyn_gather(v, idx):
    return lax.gather(v, idx[:, None], _GDN, (1,),
                      mode=lax.GatherScatterMode.PROMISE_IN_BOUNDS)


def _splat(v, j):
    return _dyn_gather(v, jnp.full((16,), j, jnp.int32))


def _lane(v, j):
    return jnp.squeeze(lax.slice(v, (j,), (j + 1,)))


def _make_sc_agg():
    mesh = plsc.VectorSubcoreMesh(core_axis_name="c", subcore_axis_name="s")

    @functools.partial(
        pl.kernel,
        mesh=mesh,
        out_type=[jax.ShapeDtypeStruct((N2 * HEADS * FW,), jnp.float32),
                  jax.ShapeDtypeStruct((N2 * HEADS * 16,), jnp.float32)],
        scratch_types=[
            pltpu.VMEM((BLK,), jnp.int32),
            pltpu.VMEM((BLK,), jnp.int32),
            pltpu.VMEM((BLK * HEADS,), jnp.float32),
            pltpu.VMEM((16,), jnp.int32),
            pltpu.VMEM((BLK, FW), jnp.float32),
            pltpu.VMEM((ACCW,), jnp.float32),
            pltpu.VMEM((DENW,), jnp.float32),
            pltpu.SemaphoreType.DMA,
        ],
    )
    def sc_agg(src_h, dst_h, w_h, bounds_h, x_h, out_h, dout_h,
               sblk_v, dblk_v, wblk_v, bnd_v, hrow_v, acc_v, den_v, sem):
        cc = lax.axis_index("c")
        ss = lax.axis_index("s")
        t = cc * 16 + ss
        lane = lax.iota(jnp.int32, 16)
        z16f = jnp.zeros((16,), jnp.float32)

        def zacc_body(i, carry):
            acc_v[pl.ds(i * 16, 16)] = z16f
            return carry

        def zden_body(i, carry):
            den_v[pl.ds(i * 16, 16)] = z16f
            return carry
        lax.fori_loop(0, ACCW // 16, zacc_body, 0)
        lax.fori_loop(0, DENW // 16, zden_body, 0)

        pltpu.sync_copy(bounds_h.at[t], bnd_v)
        bv = bnd_v[pl.ds(0, 16)]
        sts = [_lane(bv, 2 * j) for j in range(SWEEPS)]
        ens = [_lane(bv, 2 * j + 1) for j in range(SWEEPS)]

        def sweep_body(sw, carry0):
            seg = sw * NW + t
            st = sts[0]
            en = ens[0]
            for j in range(1, SWEEPS):
                st = jnp.where(sw == j, sts[j], st)
                en = jnp.where(sw == j, ens[j], en)
            lo = seg * NPT
            hi = lo + NPT

            astart = pl.multiple_of((st // 8) * 8, 8)
            nblk = (en - astart + BLK - 1) // BLK

            def blk_body(k, carry1):
                eo = pl.multiple_of(astart + k * BLK, 8)
                pltpu.sync_copy(src_h.at[pl.ds(eo, BLK)], sblk_v)
                cp = pltpu.async_copy(x_h.at[sblk_v], hrow_v, sem)
                pltpu.sync_copy(dst_h.at[pl.ds(eo, BLK)], dblk_v)
                pltpu.sync_copy(
                    w_h.at[pl.ds(eo * HEADS, BLK * HEADS)], wblk_v)
                cp.wait()

                def chunk_body(i, carry2):
                    io = i * 16
                    d16 = dblk_v[pl.ds(io, 16)]
                    ld = jnp.clip(d16, lo, hi - 1) - lo
                    pos = eo + io + lane
                    vf = jnp.where((pos >= st) & (pos < en), 1.0, 0.0)
                    wv = [wblk_v[pl.ds((io + p * 4) * HEADS, 16)]
                          for p in range(4)]
                    for q in range(16):
                        ldq = jnp.squeeze(lax.slice(ld, (q,), (q + 1,)))
                        slot = ldq * (HEADS * FW)
                        dslot = ldq * (HEADS * 16)
                        nm = _splat(vf, q)
                        wq = wv[q // 4]
                        for h_ in range(HEADS):
                            wsp = _splat(wq, (q % 4) * HEADS + h_) * nm
                            ab = pl.multiple_of(slot + h_ * FW, 16)
                            db = pl.multiple_of(dslot + h_ * 16, 16)
                            den_v[pl.ds(db, 16)] = (
                                den_v[pl.ds(db, 16)]
                                + jnp.where(lane == 0, wsp, 0.0))
                            for f in range(FW // 16):
                                hvf = hrow_v[i * 16 + q, pl.ds(f * 16, 16)]
                                acc_v[pl.ds(ab + f * 16, 16)] = (
                                    acc_v[pl.ds(ab + f * 16, 16)]
                                    + wsp * hvf)
                    return carry2
                return lax.fori_loop(0, BLK // 16, chunk_body, carry1)
            lax.fori_loop(0, nblk, blk_body, 0)

            ob = pl.multiple_of(lo * (HEADS * FW), 8192)
            pltpu.sync_copy(acc_v, out_h.at[pl.ds(ob, ACCW)])
            dob = pl.multiple_of(lo * (HEADS * 16), 2048)
            pltpu.sync_copy(den_v, dout_h.at[pl.ds(dob, DENW)])
            lax.fori_loop(0, ACCW // 16, zacc_body, 0)
            lax.fori_loop(0, DENW // 16, zden_body, 0)
            return carry0
        lax.fori_loop(0, SWEEPS, sweep_body, 0)

    return sc_agg


_sc_agg = _make_sc_agg()


def _leaky(a):
    return jnp.where(a >= 0, a, 0.2 * a)


def kernel(x, edge_index, edge_attr, batch, params):
    src = edge_index[0]
    dst = edge_index[1]

    perm = jnp.argsort(dst).astype(jnp.int32)
    dsts = dst[perm]
    srcs = src[perm]
    nb = (jnp.arange(NSEG + 1, dtype=jnp.int32) * NPT).clip(0, N)
    starts = jnp.searchsorted(dsts, nb).astype(jnp.int32)
    s1 = starts[:NSEG].reshape(SWEEPS, NW).T
    s2 = starts[1:NSEG + 1].reshape(SWEEPS, NW).T
    bounds = jnp.concatenate(
        [jnp.stack([s1, s2], axis=2).reshape(NW, 2 * SWEEPS),
         jnp.zeros((NW, 8), jnp.int32)], axis=1)
    epad = E3 - E
    srcs_p = jnp.concatenate([srcs, jnp.zeros((epad,), jnp.int32)])
    dsts_p = jnp.concatenate([dsts, jnp.full((epad,), N - 1, jnp.int32)])

    h = _linear_relu(x, params['W0'], params['b0'])

    ea_mean = jnp.mean(edge_attr, axis=0)
    outs = []
    for lp in params['layers']:
        W = lp['W'].reshape(EMB, HEADS, EMB)
        ws = jnp.einsum('dhk,hk->dh', W, lp['att_src'])
        wd = jnp.einsum('dhk,hk->dh', W, lp['att_dst'])
        ve = jnp.einsum('dhk,hk->dh',
                        lp['We'].reshape(EDGE_DIM, HEADS, EMB), lp['att_e'])

        as_ = h @ ws
        ad_ = h @ wd
        aev = edge_attr @ ve
        ae_loop = ea_mean @ ve

        a = as_[srcs] + ad_[dsts] + aev[perm]
        w_e = jnp.exp(_leaky(a))
        w_l = jnp.exp(_leaky(as_ + ad_ + ae_loop))

        w_p = jnp.concatenate(
            [w_e, jnp.zeros((epad, HEADS), jnp.float32)]).reshape(-1)
        sc_acc, sc_den = _sc_agg(srcs_p, dsts_p, w_p, bounds, h)
        acc = sc_acc.reshape(N2, HEADS, EMB)[:N]
        den = sc_den.reshape(N2, HEADS, 16)[:N, :, 0]
        acc = acc + w_l[:, :, None] * h[:, None, :]
        den = den + w_l + 1e-16
        z = acc / den[:, :, None]

        out = jnp.einsum('nhd,dhk->nk', z, W) / HEADS + lp['bias']

        m = jnp.mean(out, 0)
        v = jnp.var(out, 0)
        out = (out - m) / jnp.sqrt(v + 1e-5) * lp['gamma'] + lp['beta']
        h = h + jax.nn.relu(out)
        outs.append(h)

    seg = jax.nn.one_hot(batch, B, dtype=jnp.float32)
    pooled = [seg.T @ o for o in outs]
    zs = jnp.concatenate(pooled, axis=1)
    gates = jax.nn.softmax(zs @ params['Wg'] + params['bg'], axis=1)
    zt = jnp.stack(pooled, axis=1)
    z = jnp.sum(zt * gates[..., None], axis=1)
    return (z, outs[-1])


# speedup vs baseline: 1.8418x; 1.5232x over previous
import functools

import jax
import jax.numpy as jnp
from jax import lax
from jax.experimental import pallas as pl
from jax.experimental.pallas import tpu as pltpu
from jax.experimental.pallas import tpu_sc as plsc

N = 10000
E = 160000
IN_DIM = 128
EMB = 256
HEADS = 4
LAYERS = 3
EDGE_DIM = 16
B = 64

FW = 256
NW = 32
NPT = 88
SWEEPS = 4
N2 = NPT * NW * SWEEPS
NSEG = NW * SWEEPS
E3 = E + 1024
BLK = 128
ACCW = NPT * HEADS * FW
DENW = NPT * HEADS * 16


def _linear_relu_kernel(x_ref, w_ref, b_ref, o_ref):
    o_ref[...] = jax.nn.relu(
        jnp.dot(x_ref[...], w_ref[...], preferred_element_type=jnp.float32)
        + b_ref[...]
    )


def _linear_relu(x, w, b):
    n, k = x.shape
    m = w.shape[1]
    blk = 1000
    return pl.pallas_call(
        _linear_relu_kernel,
        out_shape=jax.ShapeDtypeStruct((n, m), jnp.float32),
        grid=(n // blk,),
        in_specs=[
            pl.BlockSpec((blk, k), lambda i: (i, 0)),
            pl.BlockSpec((k, m), lambda i: (0, 0)),
            pl.BlockSpec((m,), lambda i: (0,)),
        ],
        out_specs=pl.BlockSpec((blk, m), lambda i: (i, 0)),
    )(x, w, b)


_GDN = lax.GatherDimensionNumbers(
    offset_dims=(), collapsed_slice_dims=(0,), start_index_map=(0,))


def _dyn_gather(v, idx):
    return lax.gather(v, idx[:, None], _GDN, (1,),
                      mode=lax.GatherScatterMode.PROMISE_IN_BOUNDS)


def _splat(v, j):
    return _dyn_gather(v, jnp.full((16,), j, jnp.int32))


def _lane(v, j):
    return jnp.squeeze(lax.slice(v, (j,), (j + 1,)))


def _make_sc_agg():
    mesh = plsc.VectorSubcoreMesh(core_axis_name="c", subcore_axis_name="s")

    @functools.partial(
        pl.kernel,
        mesh=mesh,
        out_type=[jax.ShapeDtypeStruct((N2 * HEADS * FW,), jnp.float32),
                  jax.ShapeDtypeStruct((N2 * HEADS * 16,), jnp.float32)],
        scratch_types=[
            pltpu.VMEM((BLK,), jnp.int32),
            pltpu.VMEM((BLK,), jnp.int32),
            pltpu.VMEM((BLK * HEADS,), jnp.float32),
            pltpu.VMEM((16,), jnp.int32),
            pltpu.VMEM((BLK, FW), jnp.float32),
            pltpu.VMEM((ACCW,), jnp.float32),
            pltpu.VMEM((DENW,), jnp.float32),
            pltpu.SemaphoreType.DMA,
        ],
    )
    def sc_agg(src_h, dst_h, w_h, bounds_h, x_h, out_h, dout_h,
               sblk_v, dblk_v, wblk_v, bnd_v, hrow_v, acc_v, den_v, sem):
        cc = lax.axis_index("c")
        ss = lax.axis_index("s")
        t = cc * 16 + ss
        lane = lax.iota(jnp.int32, 16)
        z16f = jnp.zeros((16,), jnp.float32)

        def zacc_body(i, carry):
            acc_v[pl.ds(i * 16, 16)] = z16f
            return carry

        def zden_body(i, carry):
            den_v[pl.ds(i * 16, 16)] = z16f
            return carry
        lax.fori_loop(0, ACCW // 16, zacc_body, 0)
        lax.fori_loop(0, DENW // 16, zden_body, 0)

        pltpu.sync_copy(bounds_h.at[t], bnd_v)
        bv = bnd_v[pl.ds(0, 16)]
        sts = [_lane(bv, 2 * j) for j in range(SWEEPS)]
        ens = [_lane(bv, 2 * j + 1) for j in range(SWEEPS)]

        def sweep_body(sw, carry0):
            seg = sw * NW + t
            st = sts[0]
            en = ens[0]
            for j in range(1, SWEEPS):
                st = jnp.where(sw == j, sts[j], st)
                en = jnp.where(sw == j, ens[j], en)
            lo = seg * NPT
            hi = lo + NPT

            astart = pl.multiple_of((st // 8) * 8, 8)
            nblk = (en - astart + BLK - 1) // BLK

            def blk_body(k, carry1):
                eo = pl.multiple_of(astart + k * BLK, 8)
                pltpu.sync_copy(src_h.at[pl.ds(eo, BLK)], sblk_v)
                cp = pltpu.async_copy(x_h.at[sblk_v], hrow_v, sem)
                pltpu.sync_copy(dst_h.at[pl.ds(eo, BLK)], dblk_v)
                pltpu.sync_copy(
                    w_h.at[pl.ds(eo * HEADS, BLK * HEADS)], wblk_v)
                cp.wait()

                def chunk_body(i, carry2):
                    io = i * 16
                    d16 = dblk_v[pl.ds(io, 16)]
                    ld = jnp.clip(d16, lo, hi - 1) - lo
                    pos = eo + io + lane
                    vf = jnp.where((pos >= st) & (pos < en), 1.0, 0.0)
                    wv = [wblk_v[pl.ds((io + p * 4) * HEADS, 16)]
                          for p in range(4)]
                    for q in range(16):
                        ldq = jnp.squeeze(lax.slice(ld, (q,), (q + 1,)))
                        slot = ldq * (HEADS * FW)
                        dslot = ldq * (HEADS * 16)
                        nm = _splat(vf, q)
                        wq = wv[q // 4]
                        wsps = [
                            _splat(wq, (q % 4) * HEADS + h_) * nm
                            for h_ in range(HEADS)]
                        abs_ = [pl.multiple_of(slot + h_ * FW, 16)
                                for h_ in range(HEADS)]
                        for h_ in range(HEADS):
                            db = pl.multiple_of(dslot + h_ * 16, 16)
                            den_v[pl.ds(db, 16)] = (
                                den_v[pl.ds(db, 16)]
                                + jnp.where(lane == 0, wsps[h_], 0.0))
                        for f in range(FW // 16):
                            hvf = hrow_v[i * 16 + q, pl.ds(f * 16, 16)]
                            for h_ in range(HEADS):
                                acc_v[pl.ds(abs_[h_] + f * 16, 16)] = (
                                    acc_v[pl.ds(abs_[h_] + f * 16, 16)]
                                    + wsps[h_] * hvf)
                    return carry2
                return lax.fori_loop(0, BLK // 16, chunk_body, carry1)
            lax.fori_loop(0, nblk, blk_body, 0)

            ob = pl.multiple_of(lo * (HEADS * FW), 8192)
            pltpu.sync_copy(acc_v, out_h.at[pl.ds(ob, ACCW)])
            dob = pl.multiple_of(lo * (HEADS * 16), 2048)
            pltpu.sync_copy(den_v, dout_h.at[pl.ds(dob, DENW)])
            lax.fori_loop(0, ACCW // 16, zacc_body, 0)
            lax.fori_loop(0, DENW // 16, zden_body, 0)
            return carry0
        lax.fori_loop(0, SWEEPS, sweep_body, 0)

    return sc_agg


_sc_agg = _make_sc_agg()


def _leaky(a):
    return jnp.where(a >= 0, a, 0.2 * a)


def kernel(x, edge_index, edge_attr, batch, params):
    src = edge_index[0]
    dst = edge_index[1]

    perm = jnp.argsort(dst).astype(jnp.int32)
    dsts = dst[perm]
    srcs = src[perm]
    nb = (jnp.arange(NSEG + 1, dtype=jnp.int32) * NPT).clip(0, N)
    starts = jnp.searchsorted(dsts, nb).astype(jnp.int32)
    s1 = starts[:NSEG].reshape(SWEEPS, NW).T
    s2 = starts[1:NSEG + 1].reshape(SWEEPS, NW).T
    bounds = jnp.concatenate(
        [jnp.stack([s1, s2], axis=2).reshape(NW, 2 * SWEEPS),
         jnp.zeros((NW, 8), jnp.int32)], axis=1)
    epad = E3 - E
    srcs_p = jnp.concatenate([srcs, jnp.zeros((epad,), jnp.int32)])
    dsts_p = jnp.concatenate([dsts, jnp.full((epad,), N - 1, jnp.int32)])

    h = _linear_relu(x, params['W0'], params['b0'])

    ea_mean = jnp.mean(edge_attr, axis=0)
    outs = []
    for lp in params['layers']:
        W = lp['W'].reshape(EMB, HEADS, EMB)
        ws = jnp.einsum('dhk,hk->dh', W, lp['att_src'])
        wd = jnp.einsum('dhk,hk->dh', W, lp['att_dst'])
        ve = jnp.einsum('dhk,hk->dh',
                        lp['We'].reshape(EDGE_DIM, HEADS, EMB), lp['att_e'])

        as_ = h @ ws
        ad_ = h @ wd
        aev = edge_attr @ ve
        ae_loop = ea_mean @ ve

        a = as_[srcs] + ad_[dsts] + aev[perm]
        w_e = jnp.exp(_leaky(a))
        w_l = jnp.exp(_leaky(as_ + ad_ + ae_loop))

        w_p = jnp.concatenate(
            [w_e, jnp.zeros((epad, HEADS), jnp.float32)]).reshape(-1)
        sc_acc, sc_den = _sc_agg(srcs_p, dsts_p, w_p, bounds, h)
        acc = sc_acc.reshape(N2, HEADS, EMB)[:N]
        den = sc_den.reshape(N2, HEADS, 16)[:N, :, 0]
        acc = acc + w_l[:, :, None] * h[:, None, :]
        den = den + w_l + 1e-16
        z = acc / den[:, :, None]

        out = jnp.einsum('nhd,dhk->nk', z, W) / HEADS + lp['bias']

        m = jnp.mean(out, 0)
        v = jnp.var(out, 0)
        out = (out - m) / jnp.sqrt(v + 1e-5) * lp['gamma'] + lp['beta']
        h = h + jax.nn.relu(out)
        outs.append(h)

    seg = jax.nn.one_hot(batch, B, dtype=jnp.float32)
    pooled = [seg.T @ o for o in outs]
    zs = jnp.concatenate(pooled, axis=1)
    gates = jax.nn.softmax(zs @ params['Wg'] + params['bg'], axis=1)
    zt = jnp.stack(pooled, axis=1)
    z = jnp.sum(zt * gates[..., None], axis=1)
    return (z, outs[-1])


# speedup vs baseline: 1.9502x; 1.0588x over previous
import functools

import jax
import jax.numpy as jnp
from jax import lax
from jax.experimental import pallas as pl
from jax.experimental.pallas import tpu as pltpu
from jax.experimental.pallas import tpu_sc as plsc

N = 10000
E = 160000
IN_DIM = 128
EMB = 256
HEADS = 4
LAYERS = 3
EDGE_DIM = 16
B = 64

FW = 256
NW = 32
NPT = 80
SWEEPS = 4
N2 = NPT * NW * SWEEPS
NSEG = NW * SWEEPS
E3 = E + 1024
BLK = 160
ACCW = NPT * HEADS * FW
DENW = NPT * HEADS * 16


def _linear_relu_kernel(x_ref, w_ref, b_ref, o_ref):
    o_ref[...] = jax.nn.relu(
        jnp.dot(x_ref[...], w_ref[...], preferred_element_type=jnp.float32)
        + b_ref[...]
    )


def _linear_relu(x, w, b):
    n, k = x.shape
    m = w.shape[1]
    blk = 1000
    return pl.pallas_call(
        _linear_relu_kernel,
        out_shape=jax.ShapeDtypeStruct((n, m), jnp.float32),
        grid=(n // blk,),
        in_specs=[
            pl.BlockSpec((blk, k), lambda i: (i, 0)),
            pl.BlockSpec((k, m), lambda i: (0, 0)),
            pl.BlockSpec((m,), lambda i: (0,)),
        ],
        out_specs=pl.BlockSpec((blk, m), lambda i: (i, 0)),
    )(x, w, b)


_GDN = lax.GatherDimensionNumbers(
    offset_dims=(), collapsed_slice_dims=(0,), start_index_map=(0,))


def _dyn_gather(v, idx):
    return lax.gather(v, idx[:, None], _GDN, (1,),
                      mode=lax.GatherScatterMode.PROMISE_IN_BOUNDS)


def _splat(v, j):
    return _dyn_gather(v, jnp.full((16,), j, jnp.int32))


def _lane(v, j):
    return jnp.squeeze(lax.slice(v, (j,), (j + 1,)))


def _make_sc_agg():
    mesh = plsc.VectorSubcoreMesh(core_axis_name="c", subcore_axis_name="s")

    @functools.partial(
        pl.kernel,
        mesh=mesh,
        out_type=[jax.ShapeDtypeStruct((N2 * HEADS * FW,), jnp.float32),
                  jax.ShapeDtypeStruct((N2 * HEADS * 16,), jnp.float32)],
        scratch_types=[
            pltpu.VMEM((BLK,), jnp.int32),
            pltpu.VMEM((BLK,), jnp.int32),
            pltpu.VMEM((BLK * HEADS,), jnp.float32),
            pltpu.VMEM((16,), jnp.int32),
            pltpu.VMEM((BLK, FW), jnp.float32),
            pltpu.VMEM((ACCW,), jnp.float32),
            pltpu.VMEM((DENW,), jnp.float32),
            pltpu.SemaphoreType.DMA,
        ],
    )
    def sc_agg(src_h, dst_h, w_h, bounds_h, x_h, out_h, dout_h,
               sblk_v, dblk_v, wblk_v, bnd_v, hrow_v, acc_v, den_v, sem):
        cc = lax.axis_index("c")
        ss = lax.axis_index("s")
        t = cc * 16 + ss
        lane = lax.iota(jnp.int32, 16)
        z16f = jnp.zeros((16,), jnp.float32)

        def zacc_body(i, carry):
            acc_v[pl.ds(i * 16, 16)] = z16f
            return carry

        def zden_body(i, carry):
            den_v[pl.ds(i * 16, 16)] = z16f
            return carry
        lax.fori_loop(0, ACCW // 16, zacc_body, 0)
        lax.fori_loop(0, DENW // 16, zden_body, 0)

        pltpu.sync_copy(bounds_h.at[t], bnd_v)
        bv = bnd_v[pl.ds(0, 16)]
        sts = [_lane(bv, 2 * j) for j in range(SWEEPS)]
        ens = [_lane(bv, 2 * j + 1) for j in range(SWEEPS)]

        def sweep_body(sw, carry0):
            seg = sw * NW + t
            st = sts[0]
            en = ens[0]
            for j in range(1, SWEEPS):
                st = jnp.where(sw == j, sts[j], st)
                en = jnp.where(sw == j, ens[j], en)
            lo = seg * NPT
            hi = lo + NPT

            astart = pl.multiple_of((st // 8) * 8, 8)
            nblk = (en - astart + BLK - 1) // BLK

            def blk_body(k, carry1):
                eo = pl.multiple_of(astart + k * BLK, 8)
                pltpu.sync_copy(src_h.at[pl.ds(eo, BLK)], sblk_v)
                cp = pltpu.async_copy(x_h.at[sblk_v], hrow_v, sem)
                pltpu.sync_copy(dst_h.at[pl.ds(eo, BLK)], dblk_v)
                pltpu.sync_copy(
                    w_h.at[pl.ds(eo * HEADS, BLK * HEADS)], wblk_v)
                cp.wait()

                def chunk_body(i, carry2):
                    io = i * 16
                    d16 = dblk_v[pl.ds(io, 16)]
                    ld = jnp.clip(d16, lo, hi - 1) - lo
                    pos = eo + io + lane
                    vf = jnp.where((pos >= st) & (pos < en), 1.0, 0.0)
                    wv = [wblk_v[pl.ds((io + p * 4) * HEADS, 16)]
                          for p in range(4)]
                    for q in range(16):
                        ldq = jnp.squeeze(lax.slice(ld, (q,), (q + 1,)))
                        slot = ldq * (HEADS * FW)
                        dslot = ldq * (HEADS * 16)
                        nm = _splat(vf, q)
                        wq = wv[q // 4]
                        wsps = [
                            _splat(wq, (q % 4) * HEADS + h_) * nm
                            for h_ in range(HEADS)]
                        abs_ = [pl.multiple_of(slot + h_ * FW, 16)
                                for h_ in range(HEADS)]
                        for h_ in range(HEADS):
                            db = pl.multiple_of(dslot + h_ * 16, 16)
                            den_v[pl.ds(db, 16)] = (
                                den_v[pl.ds(db, 16)]
                                + jnp.where(lane == 0, wsps[h_], 0.0))
                        for f in range(FW // 16):
                            hvf = hrow_v[i * 16 + q, pl.ds(f * 16, 16)]
                            for h_ in range(HEADS):
                                acc_v[pl.ds(abs_[h_] + f * 16, 16)] = (
                                    acc_v[pl.ds(abs_[h_] + f * 16, 16)]
                                    + wsps[h_] * hvf)
                    return carry2
                return lax.fori_loop(0, BLK // 16, chunk_body, carry1)
            lax.fori_loop(0, nblk, blk_body, 0)

            ob = pl.multiple_of(lo * (HEADS * FW), 8192)
            pltpu.sync_copy(acc_v, out_h.at[pl.ds(ob, ACCW)])
            dob = pl.multiple_of(lo * (HEADS * 16), 2048)
            pltpu.sync_copy(den_v, dout_h.at[pl.ds(dob, DENW)])
            lax.fori_loop(0, ACCW // 16, zacc_body, 0)
            lax.fori_loop(0, DENW // 16, zden_body, 0)
            return carry0
        lax.fori_loop(0, SWEEPS, sweep_body, 0)

    return sc_agg


_sc_agg = _make_sc_agg()


def _leaky(a):
    return jnp.where(a >= 0, a, 0.2 * a)


def kernel(x, edge_index, edge_attr, batch, params):
    src = edge_index[0]
    dst = edge_index[1]

    perm = jnp.argsort(dst).astype(jnp.int32)
    dsts = dst[perm]
    srcs = src[perm]
    nb = (jnp.arange(NSEG + 1, dtype=jnp.int32) * NPT).clip(0, N)
    starts = jnp.searchsorted(dsts, nb).astype(jnp.int32)
    s1 = starts[:NSEG].reshape(SWEEPS, NW).T
    s2 = starts[1:NSEG + 1].reshape(SWEEPS, NW).T
    bounds = jnp.concatenate(
        [jnp.stack([s1, s2], axis=2).reshape(NW, 2 * SWEEPS),
         jnp.zeros((NW, 8), jnp.int32)], axis=1)
    epad = E3 - E
    srcs_p = jnp.concatenate([srcs, jnp.zeros((epad,), jnp.int32)])
    dsts_p = jnp.concatenate([dsts, jnp.full((epad,), N - 1, jnp.int32)])

    h = _linear_relu(x, params['W0'], params['b0'])

    ea_mean = jnp.mean(edge_attr, axis=0)
    outs = []
    for lp in params['layers']:
        W = lp['W'].reshape(EMB, HEADS, EMB)
        ws = jnp.einsum('dhk,hk->dh', W, lp['att_src'])
        wd = jnp.einsum('dhk,hk->dh', W, lp['att_dst'])
        ve = jnp.einsum('dhk,hk->dh',
                        lp['We'].reshape(EDGE_DIM, HEADS, EMB), lp['att_e'])

        as_ = h @ ws
        ad_ = h @ wd
        aev = edge_attr @ ve
        ae_loop = ea_mean @ ve

        a = as_[srcs] + ad_[dsts] + aev[perm]
        w_e = jnp.exp(_leaky(a))
        w_l = jnp.exp(_leaky(as_ + ad_ + ae_loop))

        w_p = jnp.concatenate(
            [w_e, jnp.zeros((epad, HEADS), jnp.float32)]).reshape(-1)
        sc_acc, sc_den = _sc_agg(srcs_p, dsts_p, w_p, bounds, h)
        acc = sc_acc.reshape(N2, HEADS, EMB)[:N]
        den = sc_den.reshape(N2, HEADS, 16)[:N, :, 0]
        acc = acc + w_l[:, :, None] * h[:, None, :]
        den = den + w_l + 1e-16
        z = acc / den[:, :, None]

        out = jnp.einsum('nhd,dhk->nk', z, W) / HEADS + lp['bias']

        m = jnp.mean(out, 0)
        v = jnp.var(out, 0)
        out = (out - m) / jnp.sqrt(v + 1e-5) * lp['gamma'] + lp['beta']
        h = h + jax.nn.relu(out)
        outs.append(h)

    seg = jax.nn.one_hot(batch, B, dtype=jnp.float32)
    pooled = [seg.T @ o for o in outs]
    zs = jnp.concatenate(pooled, axis=1)
    gates = jax.nn.softmax(zs @ params['Wg'] + params['bg'], axis=1)
    zt = jnp.stack(pooled, axis=1)
    z = jnp.sum(zt * gates[..., None], axis=1)
    return (z, outs[-1])
